# baseline (speedup 1.0000x reference)
import functools

import jax
import jax.numpy as jnp
from jax import lax
from jax.experimental import pallas as pl
from jax.experimental.pallas import tpu as pltpu
from jax.experimental.pallas import tpu_sc as plsc

NC = 2
NS = 16
NW = NC * NS
B = 4096
D = 64
L = 50
MARGIN = 1.0
RPW = B // NW
PPW = RPW // 2
KG = D // 16
NBUF = 2


def _tcf_body(uid_h, pid_h, nid_h, unbr_h, pnbr_h, nnbr_h,
              utl_h, utr_h, itl_h, itr_h,
              out_h,
              uidx_v, pidx_v, nidx_v,
              url_v, urr_v, prl_v, prr_v, nrl_v, nrr_v,
              uni_v, pni_v, nni_v,
              ubl_v, ubr_v, pbl_v, pbr_v, nbl_v, nbr_v, out_v,
              ssem, bsem):
    wid = lax.axis_index("s") * NC + lax.axis_index("c")
    base = wid * RPW
    pbase = wid * PPW

    pltpu.sync_copy(uid_h.at[pl.ds(base, RPW)], uidx_v)
    pltpu.sync_copy(pid_h.at[pl.ds(base, RPW)], pidx_v)
    pltpu.sync_copy(nid_h.at[pl.ds(base, RPW)], nidx_v)
    singles = [pltpu.async_copy(utl_h.at[uidx_v], url_v, ssem),
               pltpu.async_copy(utr_h.at[uidx_v], urr_v, ssem),
               pltpu.async_copy(itl_h.at[pidx_v], prl_v, ssem),
               pltpu.async_copy(itr_h.at[pidx_v], prr_v, ssem),
               pltpu.async_copy(itl_h.at[nidx_v], nrl_v, ssem),
               pltpu.async_copy(itr_h.at[nidx_v], nrr_v, ssem)]
    pltpu.sync_copy(unbr_h.at[pl.ds(pbase, PPW)], uni_v)
    pltpu.sync_copy(pnbr_h.at[pl.ds(pbase, PPW)], pni_v)
    pltpu.sync_copy(nnbr_h.at[pl.ds(pbase, PPW)], nni_v)

    PAIR_COPIES = ((itl_h, uni_v, ubl_v), (itr_h, uni_v, ubr_v),
                   (utl_h, pni_v, pbl_v), (utr_h, pni_v, pbr_v),
                   (utl_h, nni_v, nbl_v), (utr_h, nni_v, nbr_v))

    def start_pair(p):
        slot = lax.rem(p, NBUF)
        for tab_h, ni_v, bag_v in PAIR_COPIES:
            pltpu.async_copy(tab_h.at[ni_v.at[p]], bag_v.at[slot],
                             bsem.at[slot])

    def wait_pair(p):
        slot = lax.rem(p, NBUF)
        for tab_h, ni_v, bag_v in PAIR_COPIES:
            pltpu.make_async_copy(tab_h.at[ni_v.at[p]], bag_v.at[slot],
                                  bsem.at[slot]).wait()

    for p in range(NBUF - 1):
        start_pair(p)
    for c in singles:
        c.wait()

    inv_l = jnp.float32(1.0 / L)
    zero = jnp.zeros((16,), jnp.float32)

    def pair_body(p, acc):
        @pl.when(p + (NBUF - 1) < PPW)
        def _():
            start_pair(p + (NBUF - 1))

        wait_pair(p)
        slot = lax.rem(p, NBUF)
        HB = ((ubl_v, pbl_v, nbl_v, url_v, prl_v, nrl_v),
              (ubr_v, pbr_v, nbr_v, urr_v, prr_v, nrr_v))
        for r in range(2):
            def red(j, c):
                outs = []
                for hi, hb in enumerate(HB):
                    for t in range(3):
                        for k in range(KG // 2):
                            outs.append(c[hi * 6 + t * 2 + k]
                                        + hb[t][slot, r * L + j,
                                                pl.ds(k * 16, 16)])
                return tuple(outs)

            sums = lax.fori_loop(0, L, red, (zero,) * (3 * KG))
            row = p * 2 + r
            new = []
            for hi, hb in enumerate(HB):
                for k in range(KG // 2):
                    o = hi * 6
                    ub = sums[o + 0 * 2 + k] * inv_l
                    pb = sums[o + 1 * 2 + k] * inv_l
                    nb = sums[o + 2 * 2 + k] * inv_l
                    u = hb[3][row, pl.ds(k * 16, 16)]
                    pe = hb[4][row, pl.ds(k * 16, 16)]
                    ne = hb[5][row, pl.ds(k * 16, 16)]
                    tpos = u + ub * pb - pe
                    tneg = u + ub * nb - ne
                    v = MARGIN + tpos * tpos - tneg * tneg
                    new.append(acc[hi * 2 + k] + jnp.maximum(v, 0.0))
            acc = tuple(new)
        return acc

    acc = lax.fori_loop(0, PPW, pair_body, (zero,) * KG)
    out_v[0, :] = acc[0] + acc[1] + acc[2] + acc[3]
    pltpu.sync_copy(out_v, out_h.at[pl.ds(wid, 1)])


def kernel(user_ids, pos_ids, neg_ids, user_nbr_items, pos_item_nbr_users,
           neg_item_nbr_users, user_table, item_table):
    uid = user_ids.astype(jnp.int32)
    pid = pos_ids.astype(jnp.int32)
    nid = neg_ids.astype(jnp.int32)
    unbr = user_nbr_items.astype(jnp.int32).reshape(B // 2, 2 * L)
    pnbr = pos_item_nbr_users.astype(jnp.int32).reshape(B // 2, 2 * L)
    nnbr = neg_item_nbr_users.astype(jnp.int32).reshape(B // 2, 2 * L)
    utl = user_table[:, :D // 2]
    utr = user_table[:, D // 2:]
    itl = item_table[:, :D // 2]
    itr = item_table[:, D // 2:]

    HD = D // 2
    mesh = plsc.VectorSubcoreMesh(core_axis_name="c", subcore_axis_name="s")
    run = pl.kernel(
        _tcf_body,
        mesh=mesh,
        compiler_params=pltpu.CompilerParams(use_tc_tiling_on_sc=False),
        out_type=jax.ShapeDtypeStruct((NW, 16), jnp.float32),
        scratch_types=[
            pltpu.VMEM((RPW,), jnp.int32),
            pltpu.VMEM((RPW,), jnp.int32),
            pltpu.VMEM((RPW,), jnp.int32),
            pltpu.VMEM((RPW, HD), jnp.float32),
            pltpu.VMEM((RPW, HD), jnp.float32),
            pltpu.VMEM((RPW, HD), jnp.float32),
            pltpu.VMEM((RPW, HD), jnp.float32),
            pltpu.VMEM((RPW, HD), jnp.float32),
            pltpu.VMEM((RPW, HD), jnp.float32),
            pltpu.VMEM((PPW, 2 * L), jnp.int32),
            pltpu.VMEM((PPW, 2 * L), jnp.int32),
            pltpu.VMEM((PPW, 2 * L), jnp.int32),
            pltpu.VMEM((NBUF, 2 * L, HD), jnp.float32),
            pltpu.VMEM((NBUF, 2 * L, HD), jnp.float32),
            pltpu.VMEM((NBUF, 2 * L, HD), jnp.float32),
            pltpu.VMEM((NBUF, 2 * L, HD), jnp.float32),
            pltpu.VMEM((NBUF, 2 * L, HD), jnp.float32),
            pltpu.VMEM((NBUF, 2 * L, HD), jnp.float32),
            pltpu.VMEM((1, 16), jnp.float32),
            pltpu.SemaphoreType.DMA,
            pltpu.SemaphoreType.DMA((NBUF,)),
        ],
    )
    partials = run(uid, pid, nid, unbr, pnbr, nnbr, utl, utr, itl, itr)
    return jnp.sum(partials)


# speedup vs baseline: 2.4391x; 2.4391x over previous
import functools

import jax
import jax.numpy as jnp
from jax import lax
from jax.experimental import pallas as pl
from jax.experimental.pallas import tpu as pltpu
from jax.experimental.pallas import tpu_sc as plsc

NC = 2
NS = 16
NW = NC * NS
B = 4096
D = 64
L = 50
MARGIN = 1.0
RPW = B // NW
PPW = RPW // 2
KG = D // 16
NBUF = 2


def _tcf_body(uid_h, pid_h, nid_h, unbr_h, pnbr_h, nnbr_h, utab_h, itab_h,
              out_h,
              uidx_v, pidx_v, nidx_v, urows_v, prows_v, nrows_v,
              uni_v, pni_v, nni_v, ubag_v, pbag_v, nbag_v, out_v,
              ssem, bsem):
    wid = lax.axis_index("s") * NC + lax.axis_index("c")
    base = wid * RPW
    pbase = wid * PPW

    pltpu.sync_copy(uid_h.at[pl.ds(base, RPW)], uidx_v)
    pltpu.sync_copy(pid_h.at[pl.ds(base, RPW)], pidx_v)
    pltpu.sync_copy(nid_h.at[pl.ds(base, RPW)], nidx_v)
    cu = pltpu.async_copy(utab_h.at[uidx_v], urows_v, ssem)
    cp = pltpu.async_copy(itab_h.at[pidx_v], prows_v, ssem)
    cn = pltpu.async_copy(itab_h.at[nidx_v], nrows_v, ssem)
    pltpu.sync_copy(unbr_h.at[pl.ds(pbase, PPW)], uni_v)
    pltpu.sync_copy(pnbr_h.at[pl.ds(pbase, PPW)], pni_v)
    pltpu.sync_copy(nnbr_h.at[pl.ds(pbase, PPW)], nni_v)

    def start_pair(p):
        slot = lax.rem(p, NBUF)
        pltpu.async_copy(itab_h.at[uni_v.at[p]], ubag_v.at[slot],
                         bsem.at[slot])
        pltpu.async_copy(utab_h.at[pni_v.at[p]], pbag_v.at[slot],
                         bsem.at[slot])
        pltpu.async_copy(utab_h.at[nni_v.at[p]], nbag_v.at[slot],
                         bsem.at[slot])

    def wait_pair(p):
        slot = lax.rem(p, NBUF)
        pltpu.make_async_copy(itab_h.at[uni_v.at[p]], ubag_v.at[slot],
                              bsem.at[slot]).wait()
        pltpu.make_async_copy(utab_h.at[pni_v.at[p]], pbag_v.at[slot],
                              bsem.at[slot]).wait()
        pltpu.make_async_copy(utab_h.at[nni_v.at[p]], nbag_v.at[slot],
                              bsem.at[slot]).wait()

    for p in range(NBUF - 1):
        start_pair(p)
    cu.wait()
    cp.wait()
    cn.wait()

    inv_l = jnp.float32(1.0 / L)
    zero = jnp.zeros((16,), jnp.float32)

    def pair_body(p, acc):
        @pl.when(p + (NBUF - 1) < PPW)
        def _():
            start_pair(p + (NBUF - 1))

        wait_pair(p)
        slot = lax.rem(p, NBUF)
        for r in range(2):
            def red(j, c):
                outs = []
                for t, bag in enumerate((ubag_v, pbag_v, nbag_v)):
                    for k in range(KG):
                        outs.append(c[t * KG + k]
                                    + bag[slot, r * L + j, pl.ds(k * 16, 16)])
                return tuple(outs)

            sums = lax.fori_loop(0, L, red, (zero,) * (3 * KG))
            row = p * 2 + r
            new = []
            for k in range(KG):
                ub = sums[k] * inv_l
                pb = sums[KG + k] * inv_l
                nb = sums[2 * KG + k] * inv_l
                u = urows_v[row, pl.ds(k * 16, 16)]
                pe = prows_v[row, pl.ds(k * 16, 16)]
                ne = nrows_v[row, pl.ds(k * 16, 16)]
                tpos = u + ub * pb - pe
                tneg = u + ub * nb - ne
                v = MARGIN + tpos * tpos - tneg * tneg
                new.append(acc[k] + jnp.maximum(v, 0.0))
            acc = tuple(new)
        return acc

    acc = lax.fori_loop(0, PPW, pair_body, (zero,) * KG)
    out_v[0, :] = acc[0] + acc[1] + acc[2] + acc[3]
    pltpu.sync_copy(out_v, out_h.at[pl.ds(wid, 1)])


def kernel(user_ids, pos_ids, neg_ids, user_nbr_items, pos_item_nbr_users,
           neg_item_nbr_users, user_table, item_table):
    uid = user_ids.astype(jnp.int32)
    pid = pos_ids.astype(jnp.int32)
    nid = neg_ids.astype(jnp.int32)
    unbr = user_nbr_items.astype(jnp.int32).reshape(B // 2, 2 * L)
    pnbr = pos_item_nbr_users.astype(jnp.int32).reshape(B // 2, 2 * L)
    nnbr = neg_item_nbr_users.astype(jnp.int32).reshape(B // 2, 2 * L)

    mesh = plsc.VectorSubcoreMesh(core_axis_name="c", subcore_axis_name="s")
    run = pl.kernel(
        _tcf_body,
        mesh=mesh,
        compiler_params=pltpu.CompilerParams(use_tc_tiling_on_sc=False),
        out_type=jax.ShapeDtypeStruct((NW, 16), jnp.float32),
        scratch_types=[
            pltpu.VMEM((RPW,), jnp.int32),
            pltpu.VMEM((RPW,), jnp.int32),
            pltpu.VMEM((RPW,), jnp.int32),
            pltpu.VMEM((RPW, D), jnp.float32),
            pltpu.VMEM((RPW, D), jnp.float32),
            pltpu.VMEM((RPW, D), jnp.float32),
            pltpu.VMEM((PPW, 2 * L), jnp.int32),
            pltpu.VMEM((PPW, 2 * L), jnp.int32),
            pltpu.VMEM((PPW, 2 * L), jnp.int32),
            pltpu.VMEM((NBUF, 2 * L, D), jnp.float32),
            pltpu.VMEM((NBUF, 2 * L, D), jnp.float32),
            pltpu.VMEM((NBUF, 2 * L, D), jnp.float32),
            pltpu.VMEM((1, 16), jnp.float32),
            pltpu.SemaphoreType.DMA,
            pltpu.SemaphoreType.DMA((NBUF,)),
        ],
    )
    partials = run(uid, pid, nid, unbr, pnbr, nnbr, user_table, item_table)
    return jnp.sum(partials)
